# baseline (speedup 1.0000x reference)
import jax
import jax.numpy as jnp
from jax.experimental import pallas as pl
from jax.experimental.pallas import tpu as pltpu

_ANCHORS = ((116.0, 90.0), (156.0, 198.0), (373.0, 326.0))
_IMG = 512.0
_IGNORE_THR = 0.5
_LXY = 2.5
_LCONF = 5.0


def _body(t_ref, in_ref, out_ref, *, in_h, in_w, n_total, bs):
    b = pl.program_id(0)

    gx = t_ref[b, 0, 1] * in_w
    gy = t_ref[b, 0, 2] * in_h
    gw = t_ref[b, 0, 3] * in_w
    gh = t_ref[b, 0, 4] * in_h
    fx = jnp.floor(gx)
    fy = jnp.floor(gy)
    gi = fx.astype(jnp.int32)
    gj = fy.astype(jnp.int32)
    tx = gx - fx
    ty = gy - fy

    stride_w = _IMG / in_w
    stride_h = _IMG / in_h
    ious = []
    for aw, ah in _ANCHORS:
        aw = aw / stride_w
        ah = ah / stride_h
        inter = (jnp.maximum(jnp.minimum(gw, aw), 0.0)
                 * jnp.maximum(jnp.minimum(gh, ah), 0.0))
        union = gw * gh + aw * ah - inter + 1e-16
        ious.append(inter / union)
    best = jnp.int32(0)
    bv = ious[0]
    best = jnp.where(ious[1] > bv, jnp.int32(1), best)
    bv = jnp.maximum(bv, ious[1])
    best = jnp.where(ious[2] > bv, jnp.int32(2), best)

    rows = jax.lax.broadcasted_iota(jnp.int32, (in_h, in_w), 0)
    cols = jax.lax.broadcasted_iota(jnp.int32, (in_h, in_w), 1)
    sel = (rows == gj) & (cols == gi)

    eps = 1e-12
    top = 1.0 - 1e-12
    contrib = jnp.float32(0.0)
    noobj = jnp.float32(0.0)
    for a in range(3):
        zc = in_ref[0, 3 * a + 2, :, :]
        p = jnp.clip(jax.nn.sigmoid(zc), eps, top)
        noobj += jnp.sum(-jnp.log(1.0 - p))

        z_t = jnp.sum(jnp.where(sel, zc, 0.0))
        p_t = jnp.clip(jax.nn.sigmoid(z_t), eps, top)
        noobj -= jnp.where(ious[a] > _IGNORE_THR, -jnp.log(1.0 - p_t), 0.0)
        contrib += jnp.where(a == best, -_LCONF * jnp.log(p_t), 0.0)

        zx = jnp.sum(jnp.where(sel, in_ref[0, 3 * a + 0, :, :], 0.0))
        zy = jnp.sum(jnp.where(sel, in_ref[0, 3 * a + 1, :, :], 0.0))
        for z_v, t_v in ((zx, tx), (zy, ty)):
            p_v = jnp.clip(jax.nn.sigmoid(z_v), eps, top)
            bce = -(t_v * jnp.log(p_v) + (1.0 - t_v) * jnp.log(1.0 - p_v))
            contrib += jnp.where(a == best, _LXY * bce, 0.0)

    contrib += 0.5 * _LCONF * noobj

    @pl.when(b == 0)
    def _():
        out_ref[0, 0] = 0.0

    out_ref[0, 0] += contrib / n_total


def kernel(input, targets):
    bs, ch, in_h, in_w = input.shape
    n_total = bs * 3 * in_h * in_w
    import functools
    body = functools.partial(_body, in_h=in_h, in_w=in_w,
                             n_total=float(n_total), bs=bs)
    out = pl.pallas_call(
        body,
        grid=(bs,),
        in_specs=[
            pl.BlockSpec(targets.shape, lambda b: (0, 0, 0),
                         memory_space=pltpu.SMEM),
            pl.BlockSpec((1, ch, in_h, in_w), lambda b: (b, 0, 0, 0)),
        ],
        out_specs=pl.BlockSpec((1, 1), lambda b: (0, 0),
                               memory_space=pltpu.SMEM),
        out_shape=jax.ShapeDtypeStruct((1, 1), jnp.float32),
    )(targets, input)
    return out[0, 0]
